# baseline (speedup 1.0000x reference)
import functools

import jax
import jax.numpy as jnp
from jax import lax
from jax.experimental import pallas as pl
from jax.experimental.pallas import tpu as pltpu
from jax.experimental.pallas import tpu_sc as plsc

_NB = 8
_NW = 32
_CW = 80


def _sc_gather(table, idx):
    n = idx.shape[0]
    e = table.shape[1]
    per_w = n // _NW
    ch = per_w // _CW
    assert per_w * _NW == n and ch * _CW == per_w
    idx3 = idx.reshape(_NW, ch, _CW)
    mesh = plsc.VectorSubcoreMesh(core_axis_name="c", subcore_axis_name="s")

    @functools.partial(
        pl.kernel,
        mesh=mesh,
        out_type=jax.ShapeDtypeStruct((n, e), jnp.float32),
        scratch_types=[
            pltpu.VMEM((ch, _CW), jnp.int32),
            pltpu.VMEM((per_w, e), jnp.float32),
            pltpu.SemaphoreType.DMA,
        ],
        compiler_params=pltpu.CompilerParams(use_tc_tiling_on_sc=False),
    )
    def gather_k(table_hbm, idx_hbm, out_hbm, idx_v, rows_v, sem):
        wid = lax.axis_index("s") * 2 + lax.axis_index("c")
        pltpu.sync_copy(idx_hbm.at[wid], idx_v)
        copies = [
            pltpu.make_async_copy(
                table_hbm.at[idx_v.at[j]],
                rows_v.at[pl.ds(j * _CW, _CW)],
                sem,
            )
            for j in range(ch)
        ]
        for cp in copies:
            cp.start()
        for cp in copies:
            cp.wait()
        pltpu.sync_copy(rows_v, out_hbm.at[pl.ds(wid * per_w, per_w)])

    return gather_k(table, idx3)


def _cell(g, c, h_dim):
    i = jax.nn.sigmoid(g[:, 0:h_dim])
    f = jax.nn.sigmoid(g[:, h_dim:2 * h_dim])
    gg = jnp.tanh(g[:, 2 * h_dim:3 * h_dim])
    o = jax.nn.sigmoid(g[:, 3 * h_dim:4 * h_dim])
    c2 = f * c + i * gg
    h2 = o * jnp.tanh(c2)
    return h2, c2


def _lstm_body(x_ref, wif0, wib0, whfb0, bf0, bb0,
               wif1, wib1, whfb1, bf1, bb1,
               y_ref, h_ref, c_ref, gf, gb, y0):
    seq, bb, _ = x_ref.shape
    h_dim = whfb0.shape[0] // 2
    f32 = jnp.float32

    def run_layer(src_ref, wif, wib, whf, bf, bbias, dst_ref, slot):
        c_in = src_ref.shape[-1]
        xs = src_ref[...].reshape(seq * bb, c_in).astype(jnp.bfloat16)
        gf[...] = (jnp.dot(xs, wif[...], preferred_element_type=f32)
                   + bf[...]).reshape(seq, bb, 4 * h_dim)
        gb[...] = (jnp.dot(xs, wib[...], preferred_element_type=f32)
                   + bbias[...]).reshape(seq, bb, 4 * h_dim)
        whfb = whf[...]

        def step(t, carry):
            hf, cf, hb, cb = carry
            tb = seq - 1 - t
            hcat = jnp.concatenate([hf, hb], axis=1).astype(jnp.bfloat16)
            g_fb = jnp.dot(hcat, whfb, preferred_element_type=f32)
            g_f = gf[t] + g_fb[:, 0:4 * h_dim]
            g_b = gb[tb] + g_fb[:, 4 * h_dim:8 * h_dim]
            hf2, cf2 = _cell(g_f, cf, h_dim)
            hb2, cb2 = _cell(g_b, cb, h_dim)
            dst_ref[t, :, 0:h_dim] = hf2
            dst_ref[tb, :, h_dim:2 * h_dim] = hb2
            return hf2, cf2, hb2, cb2

        z = jnp.zeros((bb, h_dim), f32)
        hf, cf, hb, cb = lax.fori_loop(0, seq, step, (z, z, z, z))
        h_ref[slot] = hf
        h_ref[slot + 1] = hb
        c_ref[slot] = cf
        c_ref[slot + 1] = cb

    run_layer(x_ref, wif0, wib0, whfb0, bf0, bb0, y0, 0)
    run_layer(y0, wif1, wib1, whfb1, bf1, bb1, y_ref, 2)


def _run_lstm(x_tm, wp):
    seq, b, e = x_tm.shape
    h_dim = wp[2].shape[0] // 2
    bb = b // _NB
    f32 = jnp.float32

    def full(a):
        return pl.BlockSpec(a.shape, lambda i: (0,) * a.ndim)

    in_specs = [pl.BlockSpec((seq, bb, e), lambda i: (0, i, 0))]
    in_specs += [full(a) for a in wp]
    out_specs = [
        pl.BlockSpec((seq, bb, 2 * h_dim), lambda i: (0, i, 0)),
        pl.BlockSpec((4, bb, h_dim), lambda i: (0, i, 0)),
        pl.BlockSpec((4, bb, h_dim), lambda i: (0, i, 0)),
    ]
    out_shape = [
        jax.ShapeDtypeStruct((seq, b, 2 * h_dim), f32),
        jax.ShapeDtypeStruct((4, b, h_dim), f32),
        jax.ShapeDtypeStruct((4, b, h_dim), f32),
    ]
    return pl.pallas_call(
        _lstm_body,
        grid=(_NB,),
        in_specs=in_specs,
        out_specs=out_specs,
        out_shape=out_shape,
        scratch_shapes=[
            pltpu.VMEM((seq, bb, 4 * h_dim), f32),
            pltpu.VMEM((seq, bb, 4 * h_dim), f32),
            pltpu.VMEM((seq, bb, 2 * h_dim), f32),
        ],
        compiler_params=pltpu.CompilerParams(
            dimension_semantics=("arbitrary",),
            vmem_limit_bytes=120 * 1024 * 1024,
        ),
    )(x_tm, *wp)


def kernel(src, emb_W, l0f_Wih, l0f_Whh, l0f_bih, l0f_bhh,
           l0b_Wih, l0b_Whh, l0b_bih, l0b_bhh,
           l1f_Wih, l1f_Whh, l1f_bih, l1f_bhh,
           l1b_Wih, l1b_Whh, l1b_bih, l1b_bhh):
    b, seq = src.shape
    e = emb_W.shape[1]
    h_dim = l0f_Whh.shape[1]
    idx = src.astype(jnp.int32).T.reshape(-1)
    x_tm = _sc_gather(emb_W, idx).reshape(seq, b, e)
    bf16 = jnp.bfloat16

    def bd(wf, wb):
        z = jnp.zeros((h_dim, 4 * h_dim), jnp.float32)
        top = jnp.concatenate([wf.T, z], axis=1)
        bot = jnp.concatenate([z, wb.T], axis=1)
        return jnp.concatenate([top, bot], axis=0).astype(bf16)

    wp = (
        l0f_Wih.T.astype(bf16), l0b_Wih.T.astype(bf16), bd(l0f_Whh, l0b_Whh),
        (l0f_bih + l0f_bhh).reshape(1, -1), (l0b_bih + l0b_bhh).reshape(1, -1),
        l1f_Wih.T.astype(bf16), l1b_Wih.T.astype(bf16), bd(l1f_Whh, l1b_Whh),
        (l1f_bih + l1f_bhh).reshape(1, -1), (l1b_bih + l1b_bhh).reshape(1, -1),
    )
    y_tm, hs, cs = _run_lstm(x_tm, wp)
    return jnp.swapaxes(y_tm, 0, 1), (hs, cs)


# speedup vs baseline: 1.0010x; 1.0010x over previous
import functools

import jax
import jax.numpy as jnp
from jax import lax
from jax.experimental import pallas as pl
from jax.experimental.pallas import tpu as pltpu
from jax.experimental.pallas import tpu_sc as plsc

_NB = 8
_NW = 32
_CW = 80


def _sc_gather(table, idx):
    n = idx.shape[0]
    e = table.shape[1]
    per_w = n // _NW
    ch = per_w // _CW
    assert per_w * _NW == n and ch * _CW == per_w
    idx3 = idx.reshape(_NW, ch, _CW)
    mesh = plsc.VectorSubcoreMesh(core_axis_name="c", subcore_axis_name="s")

    @functools.partial(
        pl.kernel,
        mesh=mesh,
        out_type=jax.ShapeDtypeStruct((n, e), jnp.float32),
        scratch_types=[
            pltpu.VMEM((ch, _CW), jnp.int32),
            pltpu.VMEM((per_w, e), jnp.float32),
            pltpu.SemaphoreType.DMA,
        ],
        compiler_params=pltpu.CompilerParams(use_tc_tiling_on_sc=False),
    )
    def gather_k(table_hbm, idx_hbm, out_hbm, idx_v, rows_v, sem):
        wid = lax.axis_index("s") * 2 + lax.axis_index("c")
        pltpu.sync_copy(idx_hbm.at[wid], idx_v)
        copies = [
            pltpu.make_async_copy(
                table_hbm.at[idx_v.at[j]],
                rows_v.at[pl.ds(j * _CW, _CW)],
                sem,
            )
            for j in range(ch)
        ]
        for cp in copies:
            cp.start()
        for cp in copies:
            cp.wait()
        pltpu.sync_copy(rows_v, out_hbm.at[pl.ds(wid * per_w, per_w)])

    return gather_k(table, idx3)


def _cell(g, c, h_dim):
    i = jax.nn.sigmoid(g[:, 0:h_dim])
    f = jax.nn.sigmoid(g[:, h_dim:2 * h_dim])
    gg = jnp.tanh(g[:, 2 * h_dim:3 * h_dim])
    o = jax.nn.sigmoid(g[:, 3 * h_dim:4 * h_dim])
    c2 = f * c + i * gg
    h2 = o * jnp.tanh(c2)
    return h2, c2


def _lstm_body(x_ref, wif0, wib0, whfb0, bf0, bb0,
               wif1, wib1, whfb1, bf1, bb1,
               y_ref, h_ref, c_ref, gf, gb, y0):
    seq, bb, _ = x_ref.shape
    h_dim = whfb0.shape[0] // 2
    f32 = jnp.float32

    def run_layer(src_ref, wif, wib, whf, bf, bbias, dst_ref, slot):
        c_in = src_ref.shape[-1]
        xs = src_ref[...].reshape(seq * bb, c_in).astype(jnp.bfloat16)
        gf[...] = (jnp.dot(xs, wif[...], preferred_element_type=f32)
                   + bf[...]).reshape(seq, bb, 4 * h_dim).astype(gf.dtype)
        gb[...] = (jnp.dot(xs, wib[...], preferred_element_type=f32)
                   + bbias[...]).reshape(seq, bb, 4 * h_dim).astype(gb.dtype)
        whfb = whf[...]

        def step(t, carry):
            hf, cf, hb, cb = carry
            tb = seq - 1 - t
            hcat = jnp.concatenate([hf, hb], axis=1).astype(jnp.bfloat16)
            g_fb = jnp.dot(hcat, whfb, preferred_element_type=f32)
            g_f = gf[t].astype(f32) + g_fb[:, 0:4 * h_dim]
            g_b = gb[tb].astype(f32) + g_fb[:, 4 * h_dim:8 * h_dim]
            hf2, cf2 = _cell(g_f, cf, h_dim)
            hb2, cb2 = _cell(g_b, cb, h_dim)
            dst_ref[t, :, 0:h_dim] = hf2.astype(dst_ref.dtype)
            dst_ref[tb, :, h_dim:2 * h_dim] = hb2.astype(dst_ref.dtype)
            return hf2, cf2, hb2, cb2

        z = jnp.zeros((bb, h_dim), f32)
        hf, cf, hb, cb = lax.fori_loop(0, seq, step, (z, z, z, z))
        h_ref[slot] = hf
        h_ref[slot + 1] = hb
        c_ref[slot] = cf
        c_ref[slot + 1] = cb

    run_layer(x_ref, wif0, wib0, whfb0, bf0, bb0, y0, 0)
    run_layer(y0, wif1, wib1, whfb1, bf1, bb1, y_ref, 2)


def _run_lstm(x_tm, wp):
    seq, b, e = x_tm.shape
    h_dim = wp[2].shape[0] // 2
    bb = b // _NB
    f32 = jnp.float32

    def full(a):
        return pl.BlockSpec(a.shape, lambda i: (0,) * a.ndim)

    in_specs = [pl.BlockSpec((seq, bb, e), lambda i: (0, i, 0))]
    in_specs += [full(a) for a in wp]
    out_specs = [
        pl.BlockSpec((seq, bb, 2 * h_dim), lambda i: (0, i, 0)),
        pl.BlockSpec((4, bb, h_dim), lambda i: (0, i, 0)),
        pl.BlockSpec((4, bb, h_dim), lambda i: (0, i, 0)),
    ]
    out_shape = [
        jax.ShapeDtypeStruct((seq, b, 2 * h_dim), f32),
        jax.ShapeDtypeStruct((4, b, h_dim), f32),
        jax.ShapeDtypeStruct((4, b, h_dim), f32),
    ]
    return pl.pallas_call(
        _lstm_body,
        grid=(_NB,),
        in_specs=in_specs,
        out_specs=out_specs,
        out_shape=out_shape,
        scratch_shapes=[
            pltpu.VMEM((seq, bb, 4 * h_dim), jnp.bfloat16),
            pltpu.VMEM((seq, bb, 4 * h_dim), jnp.bfloat16),
            pltpu.VMEM((seq, bb, 2 * h_dim), jnp.bfloat16),
        ],
        compiler_params=pltpu.CompilerParams(
            dimension_semantics=("arbitrary",),
            vmem_limit_bytes=120 * 1024 * 1024,
        ),
    )(x_tm, *wp)


def kernel(src, emb_W, l0f_Wih, l0f_Whh, l0f_bih, l0f_bhh,
           l0b_Wih, l0b_Whh, l0b_bih, l0b_bhh,
           l1f_Wih, l1f_Whh, l1f_bih, l1f_bhh,
           l1b_Wih, l1b_Whh, l1b_bih, l1b_bhh):
    b, seq = src.shape
    e = emb_W.shape[1]
    h_dim = l0f_Whh.shape[1]
    idx = src.astype(jnp.int32).T.reshape(-1)
    x_tm = _sc_gather(emb_W, idx).reshape(seq, b, e)
    bf16 = jnp.bfloat16

    def bd(wf, wb):
        z = jnp.zeros((h_dim, 4 * h_dim), jnp.float32)
        top = jnp.concatenate([wf.T, z], axis=1)
        bot = jnp.concatenate([z, wb.T], axis=1)
        return jnp.concatenate([top, bot], axis=0).astype(bf16)

    wp = (
        l0f_Wih.T.astype(bf16), l0b_Wih.T.astype(bf16), bd(l0f_Whh, l0b_Whh),
        (l0f_bih + l0f_bhh).reshape(1, -1), (l0b_bih + l0b_bhh).reshape(1, -1),
        l1f_Wih.T.astype(bf16), l1b_Wih.T.astype(bf16), bd(l1f_Whh, l1b_Whh),
        (l1f_bih + l1f_bhh).reshape(1, -1), (l1b_bih + l1b_bhh).reshape(1, -1),
    )
    y_tm, hs, cs = _run_lstm(x_tm, wp)
    return jnp.swapaxes(y_tm, 0, 1), (hs, cs)


# speedup vs baseline: 1.0926x; 1.0916x over previous
import functools

import jax
import jax.numpy as jnp
from jax import lax
from jax.experimental import pallas as pl
from jax.experimental.pallas import tpu as pltpu
from jax.experimental.pallas import tpu_sc as plsc

_NW = 32
_CW = 80


def _sc_gather(table, idx):
    n = idx.shape[0]
    e = table.shape[1]
    per_w = n // _NW
    ch = per_w // _CW
    assert per_w * _NW == n and ch * _CW == per_w
    idx3 = idx.reshape(_NW, ch, _CW)
    mesh = plsc.VectorSubcoreMesh(core_axis_name="c", subcore_axis_name="s")

    @functools.partial(
        pl.kernel,
        mesh=mesh,
        out_type=jax.ShapeDtypeStruct((n, e), jnp.float32),
        scratch_types=[
            pltpu.VMEM((ch, _CW), jnp.int32),
            pltpu.VMEM((per_w, e), jnp.float32),
            pltpu.SemaphoreType.DMA,
        ],
        compiler_params=pltpu.CompilerParams(use_tc_tiling_on_sc=False),
    )
    def gather_k(table_hbm, idx_hbm, out_hbm, idx_v, rows_v, sem):
        wid = lax.axis_index("s") * 2 + lax.axis_index("c")
        pltpu.sync_copy(idx_hbm.at[wid], idx_v)
        copies = [
            pltpu.make_async_copy(
                table_hbm.at[idx_v.at[j]],
                rows_v.at[pl.ds(j * _CW, _CW)],
                sem,
            )
            for j in range(ch)
        ]
        for cp in copies:
            cp.start()
        for cp in copies:
            cp.wait()
        pltpu.sync_copy(rows_v, out_hbm.at[pl.ds(wid * per_w, per_w)])

    return gather_k(table, idx3)


def _cell(g, c, h_dim):
    i = jax.nn.sigmoid(g[:, 0:h_dim])
    f = jax.nn.sigmoid(g[:, h_dim:2 * h_dim])
    gg = jnp.tanh(g[:, 2 * h_dim:3 * h_dim])
    o = jax.nn.sigmoid(g[:, 3 * h_dim:4 * h_dim])
    c2 = f * c + i * gg
    h2 = o * jnp.tanh(c2)
    return h2, c2


def _lstm_body(x_ref, w0f, w0b, b0f, b0b, w1f, w1b, b1f, b1b,
               y_any, h_ref, c_ref, y0,
               stf0, stf1, stb0, stb1, semf0, semf1, semb0, semb1):
    seq, b, _ = x_ref.shape
    h_dim = h_ref.shape[-1]
    f32 = jnp.float32
    bf16 = jnp.bfloat16

    w0fv = w0f[...]
    w0bv = w0b[...]
    b0fv = b0f[...]
    b0bv = b0b[...]
    z = jnp.zeros((b, h_dim), f32)

    def l0_step(t, carry):
        hf, cf, hb, cb = carry
        tb = seq - 1 - t
        inf_ = jnp.concatenate([x_ref[t], hf.astype(bf16)], axis=1)
        inb_ = jnp.concatenate([x_ref[tb], hb.astype(bf16)], axis=1)
        g_f = jnp.dot(inf_, w0fv, preferred_element_type=f32) + b0fv
        g_b = jnp.dot(inb_, w0bv, preferred_element_type=f32) + b0bv
        hf, cf = _cell(g_f, cf, h_dim)
        hb, cb = _cell(g_b, cb, h_dim)
        y0[t, :, 0:h_dim] = hf.astype(bf16)
        y0[tb, :, h_dim:2 * h_dim] = hb.astype(bf16)
        return hf, cf, hb, cb

    hf, cf, hb, cb = lax.fori_loop(0, seq, l0_step, (z, z, z, z))
    h_ref[0] = hf
    h_ref[1] = hb
    c_ref[0] = cf
    c_ref[1] = cb

    w1fv = w1f[...]
    w1bv = w1b[...]
    b1fv = b1f[...]
    b1bv = b1b[...]

    def l1_step(k, carry):
        hf, cf, hb, cb = carry
        for par in range(2):
            t = 2 * k + par
            tb = seq - 1 - t
            inf_ = jnp.concatenate([y0[t], hf.astype(bf16)], axis=1)
            inb_ = jnp.concatenate([y0[tb], hb.astype(bf16)], axis=1)
            g_f = jnp.dot(inf_, w1fv, preferred_element_type=f32) + b1fv
            g_b = jnp.dot(inb_, w1bv, preferred_element_type=f32) + b1bv
            hf, cf = _cell(g_f, cf, h_dim)
            hb, cb = _cell(g_b, cb, h_dim)
            stf = stf0 if par == 0 else stf1
            stb = stb0 if par == 0 else stb1
            sf = semf0 if par == 0 else semf1
            sb = semb0 if par == 0 else semb1
            dst_f = y_any.at[:, pl.ds(t, 1), 0:h_dim]
            dst_b = y_any.at[:, pl.ds(tb, 1), h_dim:2 * h_dim]

            @pl.when(k > 0)
            def _():
                pltpu.make_async_copy(stf, dst_f, sf).wait()
                pltpu.make_async_copy(stb, dst_b, sb).wait()

            stf[:, 0, :] = hf
            stb[:, 0, :] = hb
            pltpu.make_async_copy(stf, dst_f, sf).start()
            pltpu.make_async_copy(stb, dst_b, sb).start()
        return hf, cf, hb, cb

    hf, cf, hb, cb = lax.fori_loop(0, seq // 2, l1_step, (z, z, z, z))
    h_ref[2] = hf
    h_ref[3] = hb
    c_ref[2] = cf
    c_ref[3] = cb

    for st, sem in ((stf0, semf0), (stf1, semf1), (stb0, semb0), (stb1, semb1)):
        pltpu.make_async_copy(st, y_any.at[:, pl.ds(0, 1), 0:h_dim], sem).wait()


def _run_lstm(x_tm, wp):
    seq, b, _ = x_tm.shape
    h_dim = wp[0].shape[1] // 4
    f32 = jnp.float32
    out_shape = [
        jax.ShapeDtypeStruct((b, seq, 2 * h_dim), f32),
        jax.ShapeDtypeStruct((4, b, h_dim), f32),
        jax.ShapeDtypeStruct((4, b, h_dim), f32),
    ]
    out_specs = [
        pl.BlockSpec(memory_space=pl.ANY),
        pl.BlockSpec(memory_space=pltpu.MemorySpace.VMEM),
        pl.BlockSpec(memory_space=pltpu.MemorySpace.VMEM),
    ]
    return pl.pallas_call(
        _lstm_body,
        out_specs=out_specs,
        out_shape=out_shape,
        scratch_shapes=[
            pltpu.VMEM((seq, b, 2 * h_dim), jnp.bfloat16),
            pltpu.VMEM((b, 1, h_dim), f32),
            pltpu.VMEM((b, 1, h_dim), f32),
            pltpu.VMEM((b, 1, h_dim), f32),
            pltpu.VMEM((b, 1, h_dim), f32),
            pltpu.SemaphoreType.DMA,
            pltpu.SemaphoreType.DMA,
            pltpu.SemaphoreType.DMA,
            pltpu.SemaphoreType.DMA,
        ],
        compiler_params=pltpu.CompilerParams(
            vmem_limit_bytes=120 * 1024 * 1024,
        ),
    )(x_tm, *wp)


def kernel(src, emb_W, l0f_Wih, l0f_Whh, l0f_bih, l0f_bhh,
           l0b_Wih, l0b_Whh, l0b_bih, l0b_bhh,
           l1f_Wih, l1f_Whh, l1f_bih, l1f_bhh,
           l1b_Wih, l1b_Whh, l1b_bih, l1b_bhh):
    b, seq = src.shape
    e = emb_W.shape[1]
    h_dim = l0f_Whh.shape[1]
    idx = src.astype(jnp.int32).T.reshape(-1)
    x_tm = _sc_gather(emb_W, idx).reshape(seq, b, e)
    x_pad = jnp.pad(x_tm, ((0, 0), (0, 0), (0, h_dim - e))).astype(jnp.bfloat16)
    bf16 = jnp.bfloat16

    def stack0(wih, whh):
        zpad = jnp.zeros((h_dim - e, 4 * h_dim), jnp.float32)
        return jnp.concatenate([wih.T, zpad, whh.T], axis=0).astype(bf16)

    def stack1(wih, whh):
        return jnp.concatenate([wih.T, whh.T], axis=0).astype(bf16)

    wp = (
        stack0(l0f_Wih, l0f_Whh), stack0(l0b_Wih, l0b_Whh),
        (l0f_bih + l0f_bhh).reshape(1, -1), (l0b_bih + l0b_bhh).reshape(1, -1),
        stack1(l1f_Wih, l1f_Whh), stack1(l1b_Wih, l1b_Whh),
        (l1f_bih + l1f_bhh).reshape(1, -1), (l1b_bih + l1b_bhh).reshape(1, -1),
    )
    y, hs, cs = _run_lstm(x_pad, wp)
    return y, (hs, cs)


# speedup vs baseline: 1.1685x; 1.0695x over previous
import functools

import jax
import jax.numpy as jnp
from jax import lax
from jax.experimental import pallas as pl
from jax.experimental.pallas import tpu as pltpu
from jax.experimental.pallas import tpu_sc as plsc

_NW = 32
_CW = 80


def _sc_gather(table, idx):
    n = idx.shape[0]
    e = table.shape[1]
    per_w = n // _NW
    ch = per_w // _CW
    assert per_w * _NW == n and ch * _CW == per_w
    idx3 = idx.reshape(_NW, ch, _CW)
    mesh = plsc.VectorSubcoreMesh(core_axis_name="c", subcore_axis_name="s")

    @functools.partial(
        pl.kernel,
        mesh=mesh,
        out_type=jax.ShapeDtypeStruct((n, e), jnp.float32),
        scratch_types=[
            pltpu.VMEM((ch, _CW), jnp.int32),
            pltpu.VMEM((per_w, e), jnp.float32),
            pltpu.SemaphoreType.DMA,
        ],
        compiler_params=pltpu.CompilerParams(use_tc_tiling_on_sc=False),
    )
    def gather_k(table_hbm, idx_hbm, out_hbm, idx_v, rows_v, sem):
        wid = lax.axis_index("s") * 2 + lax.axis_index("c")
        pltpu.sync_copy(idx_hbm.at[wid], idx_v)
        copies = [
            pltpu.make_async_copy(
                table_hbm.at[idx_v.at[j]],
                rows_v.at[pl.ds(j * _CW, _CW)],
                sem,
            )
            for j in range(ch)
        ]
        for cp in copies:
            cp.start()
        for cp in copies:
            cp.wait()
        pltpu.sync_copy(rows_v, out_hbm.at[pl.ds(wid * per_w, per_w)])

    return gather_k(table, idx3)


def _cell(g, c, h_dim):
    ti = jnp.tanh(g[:, 0:h_dim])
    tf = jnp.tanh(g[:, h_dim:2 * h_dim])
    gg = jnp.tanh(g[:, 2 * h_dim:3 * h_dim])
    to = jnp.tanh(g[:, 3 * h_dim:4 * h_dim])
    c2 = (0.5 + 0.5 * tf) * c + (0.5 + 0.5 * ti) * gg
    h2 = (0.5 + 0.5 * to) * jnp.tanh(c2)
    return h2, c2


def _lstm_body(x_ref, w0f, w0b, b0f, b0b, w1xf, w1xb, w1hf, w1hb, b1f, b1b,
               y_any, h_ref, c_ref, y0,
               stf0, stf1, stb0, stb1, semf0, semf1, semb0, semb1):
    seq, b, _ = x_ref.shape
    h_dim = h_ref.shape[-1]
    f32 = jnp.float32
    bf16 = jnp.bfloat16

    w0fv = w0f[...]
    w0bv = w0b[...]
    b0fv = b0f[...]
    b0bv = b0b[...]
    z = jnp.zeros((b, h_dim), f32)

    def l0_step(t, carry):
        hf, cf, hb, cb = carry
        tb = seq - 1 - t
        inf_ = jnp.concatenate([x_ref[t], hf.astype(bf16)], axis=1)
        inb_ = jnp.concatenate([x_ref[tb], hb.astype(bf16)], axis=1)
        g_f = jnp.dot(inf_, w0fv, preferred_element_type=f32) + b0fv
        g_b = jnp.dot(inb_, w0bv, preferred_element_type=f32) + b0bv
        hf, cf = _cell(g_f, cf, h_dim)
        hb, cb = _cell(g_b, cb, h_dim)
        y0[t, :, 0:h_dim] = hf.astype(bf16)
        y0[tb, :, h_dim:2 * h_dim] = hb.astype(bf16)
        return hf, cf, hb, cb

    hf, cf, hb, cb = lax.fori_loop(0, seq, l0_step, (z, z, z, z))
    h_ref[0] = hf
    h_ref[1] = hb
    c_ref[0] = cf
    c_ref[1] = cb

    w1xfv = w1xf[...]
    w1xbv = w1xb[...]
    w1hfv = w1hf[...]
    w1hbv = w1hb[...]
    b1fv = b1f[...]
    b1bv = b1b[...]

    def l1_step(k, carry):
        hf, cf, hb, cb = carry
        for par in range(2):
            t = 2 * k + par
            tb = seq - 1 - t
            g_f = (jnp.dot(y0[t], w1xfv, preferred_element_type=f32)
                   + jnp.dot(hf.astype(bf16), w1hfv, preferred_element_type=f32)
                   + b1fv)
            g_b = (jnp.dot(y0[tb], w1xbv, preferred_element_type=f32)
                   + jnp.dot(hb.astype(bf16), w1hbv, preferred_element_type=f32)
                   + b1bv)
            hf, cf = _cell(g_f, cf, h_dim)
            hb, cb = _cell(g_b, cb, h_dim)
            stf = stf0 if par == 0 else stf1
            stb = stb0 if par == 0 else stb1
            sf = semf0 if par == 0 else semf1
            sb = semb0 if par == 0 else semb1
            dst_f = y_any.at[:, t, 0:h_dim]
            dst_b = y_any.at[:, tb, h_dim:2 * h_dim]

            @pl.when(k > 0)
            def _():
                pltpu.make_async_copy(stf, dst_f, sf).wait()
                pltpu.make_async_copy(stb, dst_b, sb).wait()

            stf[...] = hf
            stb[...] = hb
            pltpu.make_async_copy(stf, dst_f, sf).start()
            pltpu.make_async_copy(stb, dst_b, sb).start()
        return hf, cf, hb, cb

    hf, cf, hb, cb = lax.fori_loop(0, seq // 2, l1_step, (z, z, z, z))
    h_ref[2] = hf
    h_ref[3] = hb
    c_ref[2] = cf
    c_ref[3] = cb

    for st, sem in ((stf0, semf0), (stf1, semf1), (stb0, semb0), (stb1, semb1)):
        pltpu.make_async_copy(st, y_any.at[:, 0, 0:h_dim], sem).wait()


def _run_lstm(x_tm, wp):
    seq, b, _ = x_tm.shape
    h_dim = wp[0].shape[1] // 4
    f32 = jnp.float32
    out_shape = [
        jax.ShapeDtypeStruct((b, seq, 2 * h_dim), f32),
        jax.ShapeDtypeStruct((4, b, h_dim), f32),
        jax.ShapeDtypeStruct((4, b, h_dim), f32),
    ]
    out_specs = [
        pl.BlockSpec(memory_space=pl.ANY),
        pl.BlockSpec(memory_space=pltpu.MemorySpace.VMEM),
        pl.BlockSpec(memory_space=pltpu.MemorySpace.VMEM),
    ]
    return pl.pallas_call(
        _lstm_body,
        out_specs=out_specs,
        out_shape=out_shape,
        scratch_shapes=[
            pltpu.VMEM((seq, b, 2 * h_dim), jnp.bfloat16),
            pltpu.VMEM((b, h_dim), f32),
            pltpu.VMEM((b, h_dim), f32),
            pltpu.VMEM((b, h_dim), f32),
            pltpu.VMEM((b, h_dim), f32),
            pltpu.SemaphoreType.DMA,
            pltpu.SemaphoreType.DMA,
            pltpu.SemaphoreType.DMA,
            pltpu.SemaphoreType.DMA,
        ],
        compiler_params=pltpu.CompilerParams(
            vmem_limit_bytes=120 * 1024 * 1024,
        ),
    )(x_tm, *wp)


def kernel(src, emb_W, l0f_Wih, l0f_Whh, l0f_bih, l0f_bhh,
           l0b_Wih, l0b_Whh, l0b_bih, l0b_bhh,
           l1f_Wih, l1f_Whh, l1f_bih, l1f_bhh,
           l1b_Wih, l1b_Whh, l1b_bih, l1b_bhh):
    b, seq = src.shape
    e = emb_W.shape[1]
    h_dim = l0f_Whh.shape[1]
    idx = src.astype(jnp.int32).T.reshape(-1)
    x_tm = _sc_gather(emb_W, idx).reshape(seq, b, e)
    x_pad = jnp.pad(x_tm, ((0, 0), (0, 0), (0, h_dim - e))).astype(jnp.bfloat16)
    bf16 = jnp.bfloat16

    gate_scale = jnp.concatenate([
        jnp.full((2 * h_dim,), 0.5, jnp.float32),
        jnp.ones((h_dim,), jnp.float32),
        jnp.full((h_dim,), 0.5, jnp.float32),
    ])[None, :]

    def stack0(wih, whh):
        zpad = jnp.zeros((h_dim - e, 4 * h_dim), jnp.float32)
        w = jnp.concatenate([wih.T, zpad, whh.T], axis=0)
        return (w * gate_scale).astype(bf16)

    def sw(w):
        return (w.T * gate_scale).astype(bf16)

    def sb(bih, bhh):
        return ((bih + bhh).reshape(1, -1) * gate_scale)

    wp = (
        stack0(l0f_Wih, l0f_Whh), stack0(l0b_Wih, l0b_Whh),
        sb(l0f_bih, l0f_bhh), sb(l0b_bih, l0b_bhh),
        sw(l1f_Wih), sw(l1b_Wih), sw(l1f_Whh), sw(l1b_Whh),
        sb(l1f_bih, l1f_bhh), sb(l1b_bih, l1b_bhh),
    )
    y, hs, cs = _run_lstm(x_pad, wp)
    return y, (hs, cs)


# speedup vs baseline: 1.1811x; 1.0107x over previous
import functools

import jax
import jax.numpy as jnp
from jax import lax
from jax.experimental import pallas as pl
from jax.experimental.pallas import tpu as pltpu
from jax.experimental.pallas import tpu_sc as plsc

_NW = 32
_CW = 80


def _sc_gather(table, idx):
    n = idx.shape[0]
    e = table.shape[1]
    per_w = n // _NW
    ch = per_w // _CW
    assert per_w * _NW == n and ch * _CW == per_w
    idx3 = idx.reshape(_NW, ch, _CW)
    mesh = plsc.VectorSubcoreMesh(core_axis_name="c", subcore_axis_name="s")

    @functools.partial(
        pl.kernel,
        mesh=mesh,
        out_type=jax.ShapeDtypeStruct((n, e), jnp.float32),
        scratch_types=[
            pltpu.VMEM((ch, _CW), jnp.int32),
            pltpu.VMEM((per_w, e), jnp.float32),
            pltpu.SemaphoreType.DMA,
        ],
        compiler_params=pltpu.CompilerParams(use_tc_tiling_on_sc=False),
    )
    def gather_k(table_hbm, idx_hbm, out_hbm, idx_v, rows_v, sem):
        wid = lax.axis_index("s") * 2 + lax.axis_index("c")
        pltpu.sync_copy(idx_hbm.at[wid], idx_v)
        copies = [
            pltpu.make_async_copy(
                table_hbm.at[idx_v.at[j]],
                rows_v.at[pl.ds(j * _CW, _CW)],
                sem,
            )
            for j in range(ch)
        ]
        for cp in copies:
            cp.start()
        for cp in copies:
            cp.wait()
        pltpu.sync_copy(rows_v, out_hbm.at[pl.ds(wid * per_w, per_w)])

    return gather_k(table, idx3)


def _cell(g, c, h_dim):
    ti = jnp.tanh(g[:, 0:h_dim])
    tf = jnp.tanh(g[:, h_dim:2 * h_dim])
    gg = jnp.tanh(g[:, 2 * h_dim:3 * h_dim])
    to = jnp.tanh(g[:, 3 * h_dim:4 * h_dim])
    c2 = (0.5 + 0.5 * tf) * c + (0.5 + 0.5 * ti) * gg
    h2 = (0.5 + 0.5 * to) * jnp.tanh(c2)
    return h2, c2


def _lstm_body(x_ref, w0f, w0b, b0f, b0b, w1xf, w1xb, w1hf, w1hb, b1f, b1b,
               y_any, h_ref, c_ref, y0,
               stf0, stf1, stb0, stb1, semf0, semf1, semb0, semb1):
    seq, b, _ = x_ref.shape
    h_dim = h_ref.shape[-1]
    f32 = jnp.float32
    bf16 = jnp.bfloat16

    w0fv = w0f[...]
    w0bv = w0b[...]
    b0fv = b0f[...]
    b0bv = b0b[...]
    nc = 4
    cw = b // nc
    zc = jnp.zeros((cw, h_dim), f32)
    zeros4 = tuple((zc, zc, zc, zc) for _ in range(nc))

    def l0_step(t, carry):
        tb = seq - 1 - t
        xt = x_ref[t]
        xtb = x_ref[tb]
        out = []
        for ic in range(nc):
            hf, cf, hb, cb = carry[ic]
            lo, hi = ic * cw, (ic + 1) * cw
            inf_ = jnp.concatenate([xt[lo:hi], hf.astype(bf16)], axis=1)
            inb_ = jnp.concatenate([xtb[lo:hi], hb.astype(bf16)], axis=1)
            g_f = jnp.dot(inf_, w0fv, preferred_element_type=f32) + b0fv
            g_b = jnp.dot(inb_, w0bv, preferred_element_type=f32) + b0bv
            hf, cf = _cell(g_f, cf, h_dim)
            hb, cb = _cell(g_b, cb, h_dim)
            y0[t, lo:hi, 0:h_dim] = hf.astype(bf16)
            y0[tb, lo:hi, h_dim:2 * h_dim] = hb.astype(bf16)
            out.append((hf, cf, hb, cb))
        return tuple(out)

    fin0 = lax.fori_loop(0, seq, l0_step, zeros4)
    h_ref[0] = jnp.concatenate([fin0[ic][0] for ic in range(nc)], axis=0)
    h_ref[1] = jnp.concatenate([fin0[ic][2] for ic in range(nc)], axis=0)
    c_ref[0] = jnp.concatenate([fin0[ic][1] for ic in range(nc)], axis=0)
    c_ref[1] = jnp.concatenate([fin0[ic][3] for ic in range(nc)], axis=0)

    w1xfv = w1xf[...]
    w1xbv = w1xb[...]
    w1hfv = w1hf[...]
    w1hbv = w1hb[...]
    b1fv = b1f[...]
    b1bv = b1b[...]

    def l1_step(k, carry):
        for par in range(2):
            t = 2 * k + par
            tb = seq - 1 - t
            stf = stf0 if par == 0 else stf1
            stb = stb0 if par == 0 else stb1
            sf = semf0 if par == 0 else semf1
            sb = semb0 if par == 0 else semb1
            dst_f = y_any.at[:, t, 0:h_dim]
            dst_b = y_any.at[:, tb, h_dim:2 * h_dim]

            @pl.when(k > 0)
            def _():
                pltpu.make_async_copy(stf, dst_f, sf).wait()
                pltpu.make_async_copy(stb, dst_b, sb).wait()

            yt = y0[t]
            ytb = y0[tb]
            out = []
            for ic in range(nc):
                hf, cf, hb, cb = carry[ic]
                lo, hi = ic * cw, (ic + 1) * cw
                g_f = (jnp.dot(yt[lo:hi], w1xfv, preferred_element_type=f32)
                       + jnp.dot(hf.astype(bf16), w1hfv,
                                 preferred_element_type=f32)
                       + b1fv)
                g_b = (jnp.dot(ytb[lo:hi], w1xbv, preferred_element_type=f32)
                       + jnp.dot(hb.astype(bf16), w1hbv,
                                 preferred_element_type=f32)
                       + b1bv)
                hf, cf = _cell(g_f, cf, h_dim)
                hb, cb = _cell(g_b, cb, h_dim)
                stf[lo:hi, :] = hf
                stb[lo:hi, :] = hb
                out.append((hf, cf, hb, cb))
            carry = tuple(out)
            pltpu.make_async_copy(stf, dst_f, sf).start()
            pltpu.make_async_copy(stb, dst_b, sb).start()
        return carry

    fin1 = lax.fori_loop(0, seq // 2, l1_step, zeros4)
    h_ref[2] = jnp.concatenate([fin1[ic][0] for ic in range(nc)], axis=0)
    h_ref[3] = jnp.concatenate([fin1[ic][2] for ic in range(nc)], axis=0)
    c_ref[2] = jnp.concatenate([fin1[ic][1] for ic in range(nc)], axis=0)
    c_ref[3] = jnp.concatenate([fin1[ic][3] for ic in range(nc)], axis=0)

    for st, sem in ((stf0, semf0), (stf1, semf1), (stb0, semb0), (stb1, semb1)):
        pltpu.make_async_copy(st, y_any.at[:, 0, 0:h_dim], sem).wait()


def _run_lstm(x_tm, wp):
    seq, b, _ = x_tm.shape
    h_dim = wp[0].shape[1] // 4
    f32 = jnp.float32
    out_shape = [
        jax.ShapeDtypeStruct((b, seq, 2 * h_dim), f32),
        jax.ShapeDtypeStruct((4, b, h_dim), f32),
        jax.ShapeDtypeStruct((4, b, h_dim), f32),
    ]
    out_specs = [
        pl.BlockSpec(memory_space=pl.ANY),
        pl.BlockSpec(memory_space=pltpu.MemorySpace.VMEM),
        pl.BlockSpec(memory_space=pltpu.MemorySpace.VMEM),
    ]
    return pl.pallas_call(
        _lstm_body,
        out_specs=out_specs,
        out_shape=out_shape,
        scratch_shapes=[
            pltpu.VMEM((seq, b, 2 * h_dim), jnp.bfloat16),
            pltpu.VMEM((b, h_dim), f32),
            pltpu.VMEM((b, h_dim), f32),
            pltpu.VMEM((b, h_dim), f32),
            pltpu.VMEM((b, h_dim), f32),
            pltpu.SemaphoreType.DMA,
            pltpu.SemaphoreType.DMA,
            pltpu.SemaphoreType.DMA,
            pltpu.SemaphoreType.DMA,
        ],
        compiler_params=pltpu.CompilerParams(
            vmem_limit_bytes=120 * 1024 * 1024,
        ),
    )(x_tm, *wp)


def kernel(src, emb_W, l0f_Wih, l0f_Whh, l0f_bih, l0f_bhh,
           l0b_Wih, l0b_Whh, l0b_bih, l0b_bhh,
           l1f_Wih, l1f_Whh, l1f_bih, l1f_bhh,
           l1b_Wih, l1b_Whh, l1b_bih, l1b_bhh):
    b, seq = src.shape
    e = emb_W.shape[1]
    h_dim = l0f_Whh.shape[1]
    idx = src.astype(jnp.int32).T.reshape(-1)
    x_tm = _sc_gather(emb_W, idx).reshape(seq, b, e)
    x_pad = jnp.pad(x_tm, ((0, 0), (0, 0), (0, h_dim - e))).astype(jnp.bfloat16)
    bf16 = jnp.bfloat16

    gate_scale = jnp.concatenate([
        jnp.full((2 * h_dim,), 0.5, jnp.float32),
        jnp.ones((h_dim,), jnp.float32),
        jnp.full((h_dim,), 0.5, jnp.float32),
    ])[None, :]

    def stack0(wih, whh):
        zpad = jnp.zeros((h_dim - e, 4 * h_dim), jnp.float32)
        w = jnp.concatenate([wih.T, zpad, whh.T], axis=0)
        return (w * gate_scale).astype(bf16)

    def sw(w):
        return (w.T * gate_scale).astype(bf16)

    def sb(bih, bhh):
        return ((bih + bhh).reshape(1, -1) * gate_scale)

    wp = (
        stack0(l0f_Wih, l0f_Whh), stack0(l0b_Wih, l0b_Whh),
        sb(l0f_bih, l0f_bhh), sb(l0b_bih, l0b_bhh),
        sw(l1f_Wih), sw(l1b_Wih), sw(l1f_Whh), sw(l1b_Whh),
        sb(l1f_bih, l1f_bhh), sb(l1b_bih, l1b_bhh),
    )
    y, hs, cs = _run_lstm(x_pad, wp)
    return y, (hs, cs)
